# baseline (speedup 1.0000x reference)
import functools

import jax
import jax.numpy as jnp
from jax import lax
from jax.experimental import pallas as pl
from jax.experimental.pallas import tpu as pltpu
from jax.experimental.pallas import tpu_sc as plsc

_N = 10000
_E = 320000
_F_IN = 128
_D1 = 32
_D2 = 64
_EMB = 64
_G = 64
_D = 64

_NW = 16
_GEDGE = 1024
_NGROUP = 20
_TOTG = _NW * _NGROUP
_E_PAD = _TOTG * _GEDGE
_N_PAD = 10112
_STRIPE = _N_PAD // 16

_mesh = None


def _get_mesh():
    global _mesh
    if _mesh is None:
        _mesh = plsc.VectorSubcoreMesh(core_axis_name="c", subcore_axis_name="s",
                                       num_cores=1, num_subcores=16)
    return _mesh


def _accumulate(y_hbm, idx_hbm, idxb, rows, acc, sem, base):
    def body(g, carry):
        pltpu.sync_copy(idx_hbm.at[base + g], idxb)
        pltpu.async_copy(y_hbm.at[idxb.at[0]], rows, sem).wait()
        pltpu.sync_copy(rows, acc.at[idxb.at[1]], add=True)
        return carry

    lax.fori_loop(0, _NGROUP, body, 0)


def _zero_stripe(zero_hbm, rows, acc, off):
    stage = rows.at[pl.ds(0, _STRIPE)]
    pltpu.sync_copy(zero_hbm.at[pl.ds(off, _STRIPE)], stage)
    pltpu.sync_copy(stage, acc.at[pl.ds(off, _STRIPE)])


def _copy_out_stripe(acc, rows, out_slice, off):
    stage = rows.at[pl.ds(0, _STRIPE)]
    pltpu.sync_copy(acc.at[pl.ds(off, _STRIPE)], stage)
    pltpu.sync_copy(stage, out_slice)


@functools.lru_cache(maxsize=None)
def _make_seg1():
    @functools.partial(
        pl.kernel,
        out_type=jax.ShapeDtypeStruct((_N_PAD, 2 * _D), jnp.float32),
        mesh=_get_mesh(),
        scratch_types=[
            pltpu.VMEM((2, _GEDGE), jnp.int32),
            pltpu.VMEM((_GEDGE, _D), jnp.float32),
            pltpu.VMEM_SHARED((_N_PAD, _D), jnp.float32),
            pltpu.SemaphoreType.DMA,
        ],
        compiler_params=pltpu.CompilerParams(use_tc_tiling_on_sc=False),
    )
    def seg(ylo_hbm, yhi_hbm, idx_hbm, zero_hbm, out_hbm, idxb, rows, acc, sem):
        cid = lax.axis_index("c")
        sid = lax.axis_index("s")
        wid = sid
        off = pl.multiple_of(sid * _STRIPE, 8)
        base = wid * _NGROUP
        _zero_stripe(zero_hbm, rows, acc, off)
        plsc.subcore_barrier()
        _accumulate(ylo_hbm, idx_hbm, idxb, rows, acc, sem, base)
        plsc.subcore_barrier()
        _copy_out_stripe(acc, rows,
                         out_hbm.at[pl.ds(off, _STRIPE), pl.ds(0, _D)], off)
        _zero_stripe(zero_hbm, rows, acc, off)
        plsc.subcore_barrier()
        _accumulate(yhi_hbm, idx_hbm, idxb, rows, acc, sem, base)
        plsc.subcore_barrier()
        _copy_out_stripe(acc, rows,
                         out_hbm.at[pl.ds(off, _STRIPE), pl.ds(_D, _D)], off)

    return seg


@functools.lru_cache(maxsize=None)
def _make_seg2():
    @functools.partial(
        pl.kernel,
        out_type=jax.ShapeDtypeStruct((_N_PAD, _D), jnp.float32),
        mesh=_get_mesh(),
        scratch_types=[
            pltpu.VMEM((2, _GEDGE), jnp.int32),
            pltpu.VMEM((_GEDGE, _D), jnp.float32),
            pltpu.VMEM_SHARED((_N_PAD, _D), jnp.float32),
            pltpu.SemaphoreType.DMA,
        ],
        compiler_params=pltpu.CompilerParams(use_tc_tiling_on_sc=False),
    )
    def seg(y_hbm, idx_hbm, zero_hbm, out_hbm, idxb, rows, acc, sem):
        cid = lax.axis_index("c")
        sid = lax.axis_index("s")
        wid = sid
        off = pl.multiple_of(sid * _STRIPE, 8)
        _zero_stripe(zero_hbm, rows, acc, off)
        plsc.subcore_barrier()
        _accumulate(y_hbm, idx_hbm, idxb, rows, acc, sem, wid * _NGROUP)
        plsc.subcore_barrier()
        _copy_out_stripe(acc, rows, out_hbm.at[pl.ds(off, _STRIPE)], off)

    return seg


def _mid_body(x_ref, p_ref, b1a_ref, w1a_ref, w1b_ref, b1b_ref, g1_ref,
              be1_ref, o_ref):
    t = x_ref[...] + p_ref[: _N, :]
    h = jnp.maximum(jnp.dot(t, w1a_ref[...], preferred_element_type=jnp.float32)
                    + b1a_ref[...], 0.0)
    h = jnp.dot(h, w1b_ref[...], preferred_element_type=jnp.float32) + b1b_ref[...]
    h = jnp.maximum(h, 0.0)
    mean = jnp.mean(h, axis=0, keepdims=True)
    var = jnp.mean((h - mean) * (h - mean), axis=0, keepdims=True)
    o_ref[...] = g1_ref[...] * (h - mean) / jnp.sqrt(var + 1e-5) + be1_ref[...]


def _fin_body(h_ref, p_ref, b2a_ref, w2a_ref, w2b_ref, b2b_ref, g2_ref, be2_ref,
              batch_ref, o_ref):
    t = h_ref[...] + p_ref[: _N, :]
    h = jnp.maximum(jnp.dot(t, w2a_ref[...], preferred_element_type=jnp.float32)
                    + b2a_ref[...], 0.0)
    h = jnp.dot(h, w2b_ref[...], preferred_element_type=jnp.float32) + b2b_ref[...]
    h = jnp.maximum(h, 0.0)
    mean = jnp.mean(h, axis=0, keepdims=True)
    var = jnp.mean((h - mean) * (h - mean), axis=0, keepdims=True)
    hbn = g2_ref[...] * (h - mean) / jnp.sqrt(var + 1e-5) + be2_ref[...]
    gids = lax.broadcasted_iota(jnp.int32, (_N, _G), 1)
    onehot = (batch_ref[...] == gids).astype(jnp.float32)
    sums = lax.dot_general(onehot, hbn, (((0,), (0,)), ((), ())),
                           preferred_element_type=jnp.float32,
                           precision=lax.Precision.HIGHEST)
    cnt = lax.dot_general(onehot, jnp.ones((_N, 1), jnp.float32),
                          (((0,), (0,)), ((), ())),
                          preferred_element_type=jnp.float32,
                          precision=lax.Precision.HIGHEST)
    o_ref[...] = sums / jnp.maximum(cnt, 1.0)


def kernel(x, edge_index, batch, W1a, b1a, W1b, b1b, g1, be1, W2a, b2a, W2b, b2b, g2, be2):
    pad = _E_PAD - _E
    src = jnp.concatenate([edge_index[0], jnp.zeros((pad,), jnp.int32)])
    dst = jnp.concatenate([edge_index[1], jnp.full((pad,), _N, jnp.int32)])
    idx = jnp.stack([src.reshape(_TOTG, _GEDGE), dst.reshape(_TOTG, _GEDGE)], axis=1)
    zero = jnp.zeros((_N_PAD, _D), jnp.float32)
    b2 = batch.reshape(_N, 1)
    x_lo = x[:, : _D]
    x_hi = x[:, _D:]

    p1 = _make_seg1()(x_lo, x_hi, idx, zero)

    hbn = pl.pallas_call(
        _mid_body, out_shape=jax.ShapeDtypeStruct((_N, _D2), jnp.float32)
    )(x, p1, b1a.reshape(1, _D1), W1a, W1b, b1b.reshape(1, _D2),
      g1.reshape(1, _D2), be1.reshape(1, _D2))

    p2 = _make_seg2()(hbn, idx, zero)

    out = pl.pallas_call(
        _fin_body, out_shape=jax.ShapeDtypeStruct((_G, _EMB), jnp.float32)
    )(hbn, p2, b2a.reshape(1, _D2), W2a, W2b, b2b.reshape(1, _EMB),
      g2.reshape(1, _EMB), be2.reshape(1, _EMB), b2)
    return out


# speedup vs baseline: 1.3759x; 1.3759x over previous
import functools

import jax
import jax.numpy as jnp
from jax import lax
from jax.experimental import pallas as pl
from jax.experimental.pallas import tpu as pltpu
from jax.experimental.pallas import tpu_sc as plsc

_N = 10000
_E = 320000
_F_IN = 128
_D1 = 32
_D2 = 64
_EMB = 64
_G = 64

_NW = 32
_CHUNK = 128
_NCHUNK = -(-_E // (_NW * _CHUNK))
_E_PAD = _NW * _NCHUNK * _CHUNK
_N_PAD = 10112
_STRIPE = _N_PAD // 16


@functools.lru_cache(maxsize=None)
def _make_seg_sum(d):
    mesh = plsc.VectorSubcoreMesh(core_axis_name="c", subcore_axis_name="s",
                                  num_cores=2, num_subcores=16)

    @functools.partial(
        pl.kernel,
        out_type=jax.ShapeDtypeStruct((2, _N_PAD, d), jnp.float32),
        mesh=mesh,
        scratch_types=[
            pltpu.VMEM((_CHUNK,), jnp.int32),
            pltpu.VMEM((_CHUNK,), jnp.int32),
            pltpu.VMEM((_CHUNK, d), jnp.float32),
            pltpu.VMEM_SHARED((_N_PAD, d), jnp.float32),
            pltpu.SemaphoreType.DMA,
        ],
        compiler_params=pltpu.CompilerParams(use_tc_tiling_on_sc=False),
    )
    def seg(y_hbm, src_hbm, dst_hbm, zero_hbm, out_hbm, sidx, didx, rows, acc, sem):
        cid = lax.axis_index("c")
        sid = lax.axis_index("s")
        wid = sid * 2 + cid
        off = pl.multiple_of(sid * _STRIPE, 8)
        pltpu.sync_copy(zero_hbm.at[pl.ds(off, _STRIPE)],
                        acc.at[pl.ds(off, _STRIPE)])
        plsc.subcore_barrier()

        def body(j, carry):
            pltpu.sync_copy(src_hbm.at[wid, j], sidx)
            pltpu.sync_copy(dst_hbm.at[wid, j], didx)
            pltpu.async_copy(y_hbm.at[sidx], rows, sem).wait()
            pltpu.sync_copy(rows, acc.at[didx], add=True)
            return carry

        lax.fori_loop(0, _NCHUNK, body, 0)
        plsc.subcore_barrier()
        pltpu.sync_copy(acc.at[pl.ds(off, _STRIPE)],
                        out_hbm.at[cid, pl.ds(off, _STRIPE)])

    return seg


def _mid_body(x_ref, p_ref, b1a_ref, w1a_ref, w1b_ref, b1b_ref, g1_ref, be1_ref, o_ref):
    t = x_ref[...] + p_ref[0, : _N, :] + p_ref[1, : _N, :]
    h = jnp.maximum(jnp.dot(t, w1a_ref[...], preferred_element_type=jnp.float32)
                    + b1a_ref[...], 0.0)
    h = jnp.dot(h, w1b_ref[...], preferred_element_type=jnp.float32) + b1b_ref[...]
    h = jnp.maximum(h, 0.0)
    mean = jnp.mean(h, axis=0, keepdims=True)
    var = jnp.mean((h - mean) * (h - mean), axis=0, keepdims=True)
    o_ref[...] = g1_ref[...] * (h - mean) / jnp.sqrt(var + 1e-5) + be1_ref[...]


def _fin_body(h_ref, p_ref, b2a_ref, w2a_ref, w2b_ref, b2b_ref, g2_ref, be2_ref,
              batch_ref, o_ref):
    t = h_ref[...] + p_ref[0, : _N, :] + p_ref[1, : _N, :]
    h = jnp.maximum(jnp.dot(t, w2a_ref[...], preferred_element_type=jnp.float32)
                    + b2a_ref[...], 0.0)
    h = jnp.dot(h, w2b_ref[...], preferred_element_type=jnp.float32) + b2b_ref[...]
    h = jnp.maximum(h, 0.0)
    mean = jnp.mean(h, axis=0, keepdims=True)
    var = jnp.mean((h - mean) * (h - mean), axis=0, keepdims=True)
    hbn = g2_ref[...] * (h - mean) / jnp.sqrt(var + 1e-5) + be2_ref[...]
    gids = lax.broadcasted_iota(jnp.int32, (_N, _G), 1)
    onehot = (batch_ref[...] == gids).astype(jnp.float32)
    sums = lax.dot_general(onehot, hbn, (((0,), (0,)), ((), ())),
                           preferred_element_type=jnp.float32,
                           precision=lax.Precision.HIGHEST)
    cnt = lax.dot_general(onehot, jnp.ones((_N, 1), jnp.float32),
                          (((0,), (0,)), ((), ())),
                          preferred_element_type=jnp.float32,
                          precision=lax.Precision.HIGHEST)
    o_ref[...] = sums / jnp.maximum(cnt, 1.0)


def kernel(x, edge_index, batch, W1a, b1a, W1b, b1b, g1, be1, W2a, b2a, W2b, b2b, g2, be2):
    pad = _E_PAD - _E
    src = jnp.concatenate([edge_index[0], jnp.zeros((pad,), jnp.int32)])
    dst = jnp.concatenate([edge_index[1], jnp.full((pad,), _N, jnp.int32)])
    src3 = src.reshape(_NW, _NCHUNK, _CHUNK)
    dst3 = dst.reshape(_NW, _NCHUNK, _CHUNK)
    zero128 = jnp.zeros((_N_PAD, _F_IN), jnp.float32)
    zero64 = jnp.zeros((_N_PAD, _D2), jnp.float32)
    b2 = batch.reshape(_N, 1)

    p1 = _make_seg_sum(_F_IN)(x, src3, dst3, zero128)

    hbn = pl.pallas_call(
        _mid_body, out_shape=jax.ShapeDtypeStruct((_N, _D2), jnp.float32)
    )(x, p1, b1a.reshape(1, _D1), W1a, W1b, b1b.reshape(1, _D2),
      g1.reshape(1, _D2), be1.reshape(1, _D2))

    p2 = _make_seg_sum(_D2)(hbn, src3, dst3, zero64)

    out = pl.pallas_call(
        _fin_body, out_shape=jax.ShapeDtypeStruct((_G, _EMB), jnp.float32)
    )(hbn, p2, b2a.reshape(1, _D2), W2a, W2b, b2b.reshape(1, _EMB),
      g2.reshape(1, _EMB), be2.reshape(1, _EMB), b2)
    return out


# speedup vs baseline: 1.4948x; 1.0864x over previous
import functools

import jax
import jax.numpy as jnp
from jax import lax
from jax.experimental import pallas as pl
from jax.experimental.pallas import tpu as pltpu
from jax.experimental.pallas import tpu_sc as plsc

_N = 10000
_E = 320000
_F_IN = 128
_D1 = 32
_D2 = 64
_EMB = 64
_G = 64

_NW = 32
_CHUNK = 128
_NCHUNK = -(-_E // (_NW * _CHUNK))
_E_PAD = _NW * _NCHUNK * _CHUNK
_N_PAD = 10112
_STRIPE = _N_PAD // 16


@functools.lru_cache(maxsize=None)
def _make_seg_sum(d):
    mesh = plsc.VectorSubcoreMesh(core_axis_name="c", subcore_axis_name="s",
                                  num_cores=2, num_subcores=16)

    @functools.partial(
        pl.kernel,
        out_type=jax.ShapeDtypeStruct((2, _N_PAD, d), jnp.float32),
        mesh=mesh,
        scratch_types=[
            pltpu.VMEM((2, _CHUNK), jnp.int32),
            pltpu.VMEM((_CHUNK, d), jnp.float32),
            pltpu.VMEM_SHARED((_N_PAD, d), jnp.float32),
            pltpu.SemaphoreType.DMA,
        ],
        compiler_params=pltpu.CompilerParams(use_tc_tiling_on_sc=False),
    )
    def seg(y_hbm, idx_hbm, zero_hbm, out_hbm, idxb, rows, acc, sem):
        cid = lax.axis_index("c")
        sid = lax.axis_index("s")
        wid = sid * 2 + cid
        off = pl.multiple_of(sid * _STRIPE, 8)
        pltpu.sync_copy(zero_hbm.at[pl.ds(off, _STRIPE)],
                        acc.at[pl.ds(off, _STRIPE)])
        plsc.subcore_barrier()

        def body(j, carry):
            pltpu.sync_copy(idx_hbm.at[wid, j], idxb)
            pltpu.async_copy(y_hbm.at[idxb.at[0]], rows, sem).wait()
            pltpu.sync_copy(rows, acc.at[idxb.at[1]], add=True)
            return carry

        lax.fori_loop(0, _NCHUNK, body, 0)
        plsc.subcore_barrier()
        pltpu.sync_copy(acc.at[pl.ds(off, _STRIPE)],
                        out_hbm.at[cid, pl.ds(off, _STRIPE)])

    return seg


def _mid_body(x_ref, p_ref, b1a_ref, w1a_ref, w1b_ref, b1b_ref, g1_ref, be1_ref, o_ref):
    t = x_ref[...] + p_ref[0, : _N, :] + p_ref[1, : _N, :]
    h = jnp.maximum(jnp.dot(t, w1a_ref[...], preferred_element_type=jnp.float32)
                    + b1a_ref[...], 0.0)
    h = jnp.dot(h, w1b_ref[...], preferred_element_type=jnp.float32) + b1b_ref[...]
    h = jnp.maximum(h, 0.0)
    mean = jnp.mean(h, axis=0, keepdims=True)
    var = jnp.mean((h - mean) * (h - mean), axis=0, keepdims=True)
    o_ref[...] = g1_ref[...] * (h - mean) / jnp.sqrt(var + 1e-5) + be1_ref[...]


def _fin_body(h_ref, p_ref, b2a_ref, w2a_ref, w2b_ref, b2b_ref, g2_ref, be2_ref,
              batch_ref, o_ref):
    t = h_ref[...] + p_ref[0, : _N, :] + p_ref[1, : _N, :]
    h = jnp.maximum(jnp.dot(t, w2a_ref[...], preferred_element_type=jnp.float32)
                    + b2a_ref[...], 0.0)
    h = jnp.dot(h, w2b_ref[...], preferred_element_type=jnp.float32) + b2b_ref[...]
    h = jnp.maximum(h, 0.0)
    mean = jnp.mean(h, axis=0, keepdims=True)
    var = jnp.mean((h - mean) * (h - mean), axis=0, keepdims=True)
    hbn = g2_ref[...] * (h - mean) / jnp.sqrt(var + 1e-5) + be2_ref[...]
    gids = lax.broadcasted_iota(jnp.int32, (_N, _G), 1)
    onehot = (batch_ref[...] == gids).astype(jnp.float32)
    sums = lax.dot_general(onehot, hbn, (((0,), (0,)), ((), ())),
                           preferred_element_type=jnp.float32,
                           precision=lax.Precision.HIGHEST)
    cnt = lax.dot_general(onehot, jnp.ones((_N, 1), jnp.float32),
                          (((0,), (0,)), ((), ())),
                          preferred_element_type=jnp.float32,
                          precision=lax.Precision.HIGHEST)
    o_ref[...] = sums / jnp.maximum(cnt, 1.0)


def kernel(x, edge_index, batch, W1a, b1a, W1b, b1b, g1, be1, W2a, b2a, W2b, b2b, g2, be2):
    pad = _E_PAD - _E
    src = jnp.concatenate([edge_index[0], jnp.zeros((pad,), jnp.int32)])
    dst = jnp.concatenate([edge_index[1], jnp.full((pad,), _N, jnp.int32)])
    idx3 = jnp.stack([src.reshape(_NW, _NCHUNK, _CHUNK),
                      dst.reshape(_NW, _NCHUNK, _CHUNK)], axis=2)
    zero128 = jnp.zeros((_N_PAD, _F_IN), jnp.float32)
    zero64 = jnp.zeros((_N_PAD, _D2), jnp.float32)
    b2 = batch.reshape(_N, 1)

    p1 = _make_seg_sum(_F_IN)(x, idx3, zero128)

    hbn = pl.pallas_call(
        _mid_body, out_shape=jax.ShapeDtypeStruct((_N, _D2), jnp.float32)
    )(x, p1, b1a.reshape(1, _D1), W1a, W1b, b1b.reshape(1, _D2),
      g1.reshape(1, _D2), be1.reshape(1, _D2))

    p2 = _make_seg_sum(_D2)(hbn, idx3, zero64)

    out = pl.pallas_call(
        _fin_body, out_shape=jax.ShapeDtypeStruct((_G, _EMB), jnp.float32)
    )(hbn, p2, b2a.reshape(1, _D2), W2a, W2b, b2b.reshape(1, _EMB),
      g2.reshape(1, _EMB), be2.reshape(1, _EMB), b2)
    return out
